# baseline (speedup 1.0000x reference)
import functools

import jax
import jax.numpy as jnp
from jax import lax
from jax.experimental import pallas as pl
from jax.experimental.pallas import tpu as pltpu
from jax.experimental.pallas import tpu_sc as plsc

NUM_CORES = 2
NUM_SUBCORES = 16
NW = NUM_CORES * NUM_SUBCORES
LANES = 16
B = 16384
D = 32
BPW = B // NW
CHUNK = 128
NCHUNK = BPW // CHUNK

_mesh = plsc.VectorSubcoreMesh(core_axis_name="c", subcore_axis_name="s")


@functools.partial(
    pl.kernel,
    mesh=_mesh,
    compiler_params=pltpu.CompilerParams(
        needs_layout_passes=False, use_tc_tiling_on_sc=False),
    out_type=jax.ShapeDtypeStruct((B,), jnp.float32),
    scratch_types=[
        pltpu.VMEM((NCHUNK, CHUNK), jnp.int32),
        pltpu.VMEM((NCHUNK, CHUNK), jnp.int32),
        pltpu.VMEM((BPW, D), jnp.float32),
        pltpu.VMEM((BPW, D), jnp.float32),
        pltpu.VMEM((LANES,), jnp.float32),
        pltpu.VMEM((BPW,), jnp.float32),
        pltpu.SemaphoreType.DMA,
        pltpu.SemaphoreType.DMA,
    ],
)
def _svd_score(uid_hbm, iid_hbm, ut_hbm, it_hbm, bias_hbm, out_hbm,
               uidv, iidv, urows, irows, biasv, outv, sem_u, sem_i):
    wid = lax.axis_index("s") * NUM_CORES + lax.axis_index("c")
    base = wid * BPW

    pltpu.sync_copy(uid_hbm.at[wid], uidv)
    pltpu.sync_copy(iid_hbm.at[wid], iidv)
    pltpu.sync_copy(bias_hbm, biasv)

    copies = []
    for j in range(NCHUNK):
        copies.append(pltpu.async_copy(
            ut_hbm.at[uidv.at[j]], urows.at[pl.ds(j * CHUNK, CHUNK)], sem_u))
        copies.append(pltpu.async_copy(
            it_hbm.at[iidv.at[j]], irows.at[pl.ds(j * CHUNK, CHUNK)], sem_i))
    for c in copies:
        c.wait()

    bias = biasv[...]

    def g_body(g, carry):
        row = g * LANES + lax.iota(jnp.int32, LANES)
        acc = jnp.zeros((LANES,), jnp.float32)
        for d in range(D):
            col = jnp.full((LANES,), d, jnp.int32)
            uu = plsc.load_gather(urows, [row, col])
            ii = plsc.load_gather(irows, [row, col])
            acc = acc + uu * ii
        outv[pl.ds(g * LANES, LANES)] = acc + bias
        return carry

    lax.fori_loop(0, BPW // LANES, g_body, 0)

    pltpu.sync_copy(outv, out_hbm.at[pl.ds(base, BPW)])


def kernel(user_ids, item_ids, user_table, item_table, user_bias, item_bias):
    uid = user_ids.astype(jnp.int32).reshape(NW, NCHUNK, CHUNK)
    iid = item_ids.astype(jnp.int32).reshape(NW, NCHUNK, CHUNK)
    bias16 = jnp.broadcast_to(
        (3.5 + user_bias + item_bias).astype(jnp.float32), (LANES,))
    score = _svd_score(uid, iid, user_table, item_table, bias16)
    return score.reshape(B, 1)


# speedup vs baseline: 3.4459x; 3.4459x over previous
import functools

import jax
import jax.numpy as jnp
from jax import lax
from jax.experimental import pallas as pl
from jax.experimental.pallas import tpu as pltpu
from jax.experimental.pallas import tpu_sc as plsc

NUM_CORES = 2
NUM_SUBCORES = 16
NW = NUM_CORES * NUM_SUBCORES
LANES = 16
B = 16384
D = 32
V = 1000000
EPW = B // NW
GRP = 8
NPH = EPW // GRP

_mesh = plsc.VectorSubcoreMesh(core_axis_name="c", subcore_axis_name="s")


@functools.partial(
    pl.kernel,
    mesh=_mesh,
    compiler_params=pltpu.CompilerParams(
        needs_layout_passes=False, use_tc_tiling_on_sc=True),
    out_type=jax.ShapeDtypeStruct((B,), jnp.float32),
    scratch_types=[
        pltpu.VMEM((EPW + LANES,), jnp.int32),
        pltpu.VMEM((EPW + LANES,), jnp.int32),
        pltpu.VMEM((GRP, D, 128), jnp.float32),
        pltpu.VMEM((GRP, D, 128), jnp.float32),
        pltpu.VMEM((LANES,), jnp.float32),
        pltpu.VMEM((EPW,), jnp.float32),
        pltpu.SemaphoreType.DMA,
        pltpu.SemaphoreType.DMA,
    ],
)
def _svd_score(uid_hbm, iid_hbm, ut_hbm, it_hbm, bias_hbm, out_hbm,
               uidv, iidv, ublk, iblk, biasv, outv, semu, semi):
    wid = lax.axis_index("s") * NUM_CORES + lax.axis_index("c")
    base = wid * EPW

    pltpu.sync_copy(uid_hbm.at[wid], uidv)
    pltpu.sync_copy(iid_hbm.at[wid], iidv)
    pltpu.sync_copy(bias_hbm, biasv)

    lane = lax.iota(jnp.int32, LANES)
    slotv = lane & (GRP - 1)
    lo_mask = lane < GRP
    bias = biasv[...]

    def phase(uvec, ivec, lane_off):
        copies = []
        for l in range(GRP):
            ub = pl.multiple_of((uvec[lane_off + l] >> 7) << 7, 128)
            copies.append(pltpu.async_copy(
                ut_hbm.at[:, pl.ds(ub, 128)], ublk.at[l], semu))
            ib = pl.multiple_of((ivec[lane_off + l] >> 7) << 7, 128)
            copies.append(pltpu.async_copy(
                it_hbm.at[:, pl.ds(ib, 128)], iblk.at[l], semi))
        for c in copies:
            c.wait()
        ulane = uvec & 127
        ilane = ivec & 127
        acc = jnp.zeros((LANES,), jnp.float32)
        for d in range(D):
            dvec = jnp.full((LANES,), d, jnp.int32)
            uu = plsc.load_gather(ublk, [slotv, dvec, ulane])
            ii = plsc.load_gather(iblk, [slotv, dvec, ilane])
            acc = acc + uu * ii
        return acc

    def body(k, carry):
        uvec = uidv[pl.ds(k * LANES, LANES)]
        ivec = iidv[pl.ds(k * LANES, LANES)]
        acc_a = phase(uvec, ivec, 0)
        acc_b = phase(uvec, ivec, GRP)
        res = jnp.where(lo_mask, acc_a, acc_b) + bias
        outv[pl.ds(k * LANES, LANES)] = res
        return carry

    lax.fori_loop(0, EPW // LANES, body, 0)

    pltpu.sync_copy(outv, out_hbm.at[pl.ds(base, EPW)])


def kernel(user_ids, item_ids, user_table, item_table, user_bias, item_bias):
    uid = user_ids.astype(jnp.int32).reshape(NW, EPW)
    iid = item_ids.astype(jnp.int32).reshape(NW, EPW)
    bias16 = jnp.broadcast_to(
        (3.5 + user_bias + item_bias).astype(jnp.float32), (LANES,))
    score = _svd_score(uid, iid, user_table.T, item_table.T, bias16)
    return score.reshape(B, 1)


# speedup vs baseline: 3.9510x; 1.1466x over previous
import functools

import jax
import jax.numpy as jnp
from jax import lax
from jax.experimental import pallas as pl
from jax.experimental.pallas import tpu as pltpu
from jax.experimental.pallas import tpu_sc as plsc

NUM_CORES = 2
NUM_SUBCORES = 16
NW = NUM_CORES * NUM_SUBCORES
LANES = 16
B = 16384
D = 32
EPW = B // NW
GRP = 4
NK = EPW // LANES

_mesh = plsc.VectorSubcoreMesh(core_axis_name="c", subcore_axis_name="s")


@functools.partial(
    pl.kernel,
    mesh=_mesh,
    compiler_params=pltpu.CompilerParams(
        needs_layout_passes=False, use_tc_tiling_on_sc=True),
    out_type=jax.ShapeDtypeStruct((B,), jnp.float32),
    scratch_types=[
        pltpu.VMEM((EPW,), jnp.int32),
        pltpu.VMEM((EPW,), jnp.int32),
        pltpu.VMEM((2, GRP, D, 128), jnp.float32),
        pltpu.VMEM((2, GRP, D, 128), jnp.float32),
        pltpu.VMEM((LANES,), jnp.float32),
        pltpu.VMEM((EPW,), jnp.float32),
        pltpu.SemaphoreType.DMA,
        pltpu.SemaphoreType.DMA,
    ],
)
def _svd_score(uid_hbm, iid_hbm, ut_hbm, it_hbm, bias_hbm, out_hbm,
               uidv, iidv, ublk, iblk, biasv, outv, semu, semi):
    wid = lax.axis_index("s") * NUM_CORES + lax.axis_index("c")
    base = wid * EPW

    pltpu.sync_copy(uid_hbm.at[wid], uidv)
    pltpu.sync_copy(iid_hbm.at[wid], iidv)
    pltpu.sync_copy(bias_hbm, biasv)

    lane = lax.iota(jnp.int32, LANES)
    slotv = lane & (GRP - 1)
    qmask = [((lane >> 2) == q).astype(jnp.float32) for q in range(4)]
    bias = biasv[...]

    def fire(uvec, ivec, lane_off, buf):
        for l in range(GRP):
            ub = pl.multiple_of((uvec[lane_off + l] >> 7) << 7, 128)
            pltpu.async_copy(ut_hbm.at[:, pl.ds(ub, 128)], ublk.at[buf, l],
                             semu)
            ib = pl.multiple_of((ivec[lane_off + l] >> 7) << 7, 128)
            pltpu.async_copy(it_hbm.at[:, pl.ds(ib, 128)], iblk.at[buf, l],
                             semi)

    def drain(buf):
        for l in range(GRP):
            pltpu.make_async_copy(
                ut_hbm.at[:, pl.ds(0, 128)], ublk.at[buf, l], semu).wait()
            pltpu.make_async_copy(
                it_hbm.at[:, pl.ds(0, 128)], iblk.at[buf, l], semi).wait()

    def comp(uvec, ivec, buf):
        bufv = jnp.full((LANES,), buf, jnp.int32)
        ulane = uvec & 127
        ilane = ivec & 127
        acc = jnp.zeros((LANES,), jnp.float32)
        for d in range(D):
            dvec = jnp.full((LANES,), d, jnp.int32)
            uu = plsc.load_gather(ublk, [bufv, slotv, dvec, ulane])
            ii = plsc.load_gather(iblk, [bufv, slotv, dvec, ilane])
            acc = acc + uu * ii
        return acc

    uvec0 = uidv[pl.ds(0, LANES)]
    ivec0 = iidv[pl.ds(0, LANES)]
    fire(uvec0, ivec0, 0, 0)

    def body(k, carry):
        uvec = uidv[pl.ds(k * LANES, LANES)]
        ivec = iidv[pl.ds(k * LANES, LANES)]
        fire(uvec, ivec, 4, 1)
        drain(0)
        a0 = comp(uvec, ivec, 0)
        fire(uvec, ivec, 8, 0)
        drain(1)
        a1 = comp(uvec, ivec, 1)
        fire(uvec, ivec, 12, 1)
        drain(0)
        a2 = comp(uvec, ivec, 0)

        @pl.when(k < NK - 1)
        def _fire_next():
            nvec_u = uidv[pl.ds((k + 1) * LANES, LANES)]
            nvec_i = iidv[pl.ds((k + 1) * LANES, LANES)]
            fire(nvec_u, nvec_i, 0, 0)

        drain(1)
        a3 = comp(uvec, ivec, 1)
        res = (a0 * qmask[0] + a1 * qmask[1] + a2 * qmask[2] + a3 * qmask[3]
               + bias)
        outv[pl.ds(k * LANES, LANES)] = res
        return carry

    lax.fori_loop(0, NK, body, 0)

    pltpu.sync_copy(outv, out_hbm.at[pl.ds(base, EPW)])


def kernel(user_ids, item_ids, user_table, item_table, user_bias, item_bias):
    uid = user_ids.astype(jnp.int32).reshape(NW, EPW)
    iid = item_ids.astype(jnp.int32).reshape(NW, EPW)
    bias16 = jnp.broadcast_to(
        (3.5 + user_bias + item_bias).astype(jnp.float32), (LANES,))
    score = _svd_score(uid, iid, user_table.T, item_table.T, bias16)
    return score.reshape(B, 1)


# speedup vs baseline: 4.0564x; 1.0267x over previous
import functools

import jax
import jax.numpy as jnp
from jax import lax
from jax.experimental import pallas as pl
from jax.experimental.pallas import tpu as pltpu
from jax.experimental.pallas import tpu_sc as plsc

NUM_CORES = 2
NUM_SUBCORES = 16
NW = NUM_CORES * NUM_SUBCORES
LANES = 16
B = 16384
D = 32
EPW = B // NW
GRP = 2
NBUF = 4
NK = EPW // LANES

_mesh = plsc.VectorSubcoreMesh(core_axis_name="c", subcore_axis_name="s")


@functools.partial(
    pl.kernel,
    mesh=_mesh,
    compiler_params=pltpu.CompilerParams(
        needs_layout_passes=False, use_tc_tiling_on_sc=True),
    out_type=jax.ShapeDtypeStruct((B,), jnp.float32),
    scratch_types=[
        pltpu.VMEM((EPW,), jnp.int32),
        pltpu.VMEM((EPW,), jnp.int32),
        pltpu.VMEM((NBUF, GRP, D, 128), jnp.float32),
        pltpu.VMEM((NBUF, GRP, D, 128), jnp.float32),
        pltpu.VMEM((LANES,), jnp.float32),
        pltpu.VMEM((EPW,), jnp.float32),
        pltpu.SemaphoreType.DMA,
        pltpu.SemaphoreType.DMA,
    ],
)
def _svd_score(uid_hbm, iid_hbm, ut_hbm, it_hbm, bias_hbm, out_hbm,
               uidv, iidv, ublk, iblk, biasv, outv, semu, semi):
    wid = lax.axis_index("s") * NUM_CORES + lax.axis_index("c")
    base = wid * EPW

    pltpu.sync_copy(uid_hbm.at[wid], uidv)
    pltpu.sync_copy(iid_hbm.at[wid], iidv)
    pltpu.sync_copy(bias_hbm, biasv)

    lane = lax.iota(jnp.int32, LANES)
    slotv = lane & (GRP - 1)
    pmask = [((lane >> 1) == p).astype(jnp.float32) for p in range(8)]
    bias = biasv[...]

    def fire(uvec, ivec, lane_off, buf):
        for l in range(GRP):
            ub = pl.multiple_of((uvec[lane_off + l] >> 7) << 7, 128)
            pltpu.async_copy(ut_hbm.at[:, pl.ds(ub, 128)], ublk.at[buf, l],
                             semu)
            ib = pl.multiple_of((ivec[lane_off + l] >> 7) << 7, 128)
            pltpu.async_copy(it_hbm.at[:, pl.ds(ib, 128)], iblk.at[buf, l],
                             semi)

    def drain(buf):
        for l in range(GRP):
            pltpu.make_async_copy(
                ut_hbm.at[:, pl.ds(0, 128)], ublk.at[buf, l], semu).wait()
            pltpu.make_async_copy(
                it_hbm.at[:, pl.ds(0, 128)], iblk.at[buf, l], semi).wait()

    def comp(uvec, ivec, buf):
        bufv = jnp.full((LANES,), buf, jnp.int32)
        ulane = uvec & 127
        ilane = ivec & 127
        acc = jnp.zeros((LANES,), jnp.float32)
        for d in range(D):
            dvec = jnp.full((LANES,), d, jnp.int32)
            uu = plsc.load_gather(ublk, [bufv, slotv, dvec, ulane])
            ii = plsc.load_gather(iblk, [bufv, slotv, dvec, ilane])
            acc = acc + uu * ii
        return acc

    uvec0 = uidv[pl.ds(0, LANES)]
    ivec0 = iidv[pl.ds(0, LANES)]
    for p in range(3):
        fire(uvec0, ivec0, p * GRP, p)

    def body(k, carry):
        uvec = uidv[pl.ds(k * LANES, LANES)]
        ivec = iidv[pl.ds(k * LANES, LANES)]
        nvec_u = uidv[pl.ds(jnp.minimum(k + 1, NK - 1) * LANES, LANES)]
        nvec_i = iidv[pl.ds(jnp.minimum(k + 1, NK - 1) * LANES, LANES)]
        accs = []
        for p in range(8):
            nxt = p + 3
            if nxt < 8:
                fire(uvec, ivec, nxt * GRP, nxt & 3)
            else:
                @pl.when(k < NK - 1)
                def _fire_next(nxt=nxt):
                    fire(nvec_u, nvec_i, (nxt - 8) * GRP, nxt & 3)

            drain(p & 3)
            accs.append(comp(uvec, ivec, p & 3))
        res = bias
        for p in range(8):
            res = res + accs[p] * pmask[p]
        outv[pl.ds(k * LANES, LANES)] = res
        return carry

    lax.fori_loop(0, NK, body, 0)

    pltpu.sync_copy(outv, out_hbm.at[pl.ds(base, EPW)])


def kernel(user_ids, item_ids, user_table, item_table, user_bias, item_bias):
    uid = user_ids.astype(jnp.int32).reshape(NW, EPW)
    iid = item_ids.astype(jnp.int32).reshape(NW, EPW)
    bias16 = jnp.broadcast_to(
        (3.5 + user_bias + item_bias).astype(jnp.float32), (LANES,))
    score = _svd_score(uid, iid, user_table.T, item_table.T, bias16)
    return score.reshape(B, 1)
